# Initial kernel scaffold; baseline (speedup 1.0000x reference)
import jax
import jax.numpy as jnp
from jax.experimental import pallas as pl


def kernel(ids, ori_weight, think_weight):
    raise NotImplementedError("write your pallas kernel here")


# speedup vs baseline: 6.5662x; 6.5662x over previous
import functools

import jax
import jax.numpy as jnp
from jax import lax
from jax.experimental import pallas as pl
from jax.experimental.pallas import tpu as pltpu
from jax.experimental.pallas import tpu_sc as plsc

EMBED_DIM = 64
CHUNK = 512


def _make_gather(n_ids: int):
    info = plsc.get_sparse_core_info()
    nw = info.num_cores * info.num_subcores
    assert n_ids % (nw * CHUNK) == 0
    b_per_w = n_ids // nw
    n_chunks = b_per_w // CHUNK
    mesh = plsc.VectorSubcoreMesh(core_axis_name="c", subcore_axis_name="s")

    @functools.partial(
        pl.kernel,
        mesh=mesh,
        out_type=jax.ShapeDtypeStruct((n_ids, EMBED_DIM), jnp.float32),
        scratch_types=[
            pltpu.VMEM((CHUNK,), jnp.int32),
            pltpu.VMEM((CHUNK, EMBED_DIM), jnp.float32),
            pltpu.SemaphoreType.DMA,
        ],
        compiler_params=pltpu.CompilerParams(use_tc_tiling_on_sc=False),
    )
    def gather_kernel(ids_hbm, table_hbm, out_hbm, idx_v, rows_v, sem):
        wid = lax.axis_index("s") * info.num_cores + lax.axis_index("c")
        base = wid * b_per_w

        def chunk_body(g, carry):
            off = base + g * CHUNK
            pltpu.sync_copy(ids_hbm.at[pl.ds(off, CHUNK)], idx_v)
            pltpu.async_copy(table_hbm.at[idx_v], rows_v, sem).wait()
            pltpu.sync_copy(rows_v, out_hbm.at[pl.ds(off, CHUNK)])
            return carry

        lax.fori_loop(0, n_chunks, chunk_body, 0)

    return gather_kernel


def kernel(ids, ori_weight, think_weight):
    table = jnp.concatenate([ori_weight, think_weight], axis=0)
    flat_ids = ids.reshape(-1).astype(jnp.int32)
    out = _make_gather(flat_ids.shape[0])(flat_ids, table)
    return out.reshape(ids.shape + (EMBED_DIM,))


# speedup vs baseline: 6.8589x; 1.0446x over previous
import functools

import jax
import jax.numpy as jnp
from jax import lax
from jax.experimental import pallas as pl
from jax.experimental.pallas import tpu as pltpu
from jax.experimental.pallas import tpu_sc as plsc

EMBED_DIM = 64
CHUNK = 512


def _make_gather(n_ids: int):
    info = plsc.get_sparse_core_info()
    nw = info.num_cores * info.num_subcores
    assert n_ids % (nw * 2 * CHUNK) == 0
    b_per_w = n_ids // nw
    n_chunks = b_per_w // CHUNK
    mesh = plsc.VectorSubcoreMesh(core_axis_name="c", subcore_axis_name="s")

    @functools.partial(
        pl.kernel,
        mesh=mesh,
        out_type=jax.ShapeDtypeStruct((n_ids, EMBED_DIM), jnp.float32),
        scratch_types=[
            pltpu.VMEM((2, CHUNK), jnp.int32),
            pltpu.VMEM((2, CHUNK, EMBED_DIM), jnp.float32),
            pltpu.SemaphoreType.DMA,
            pltpu.SemaphoreType.DMA,
        ],
        compiler_params=pltpu.CompilerParams(use_tc_tiling_on_sc=False),
    )
    def gather_kernel(ids_hbm, table_hbm, out_hbm, idx_v, rows_v, gsem, wsem):
        wid = lax.axis_index("s") * info.num_cores + lax.axis_index("c")
        base = wid * b_per_w

        def fetch(g, b):
            pltpu.sync_copy(
                ids_hbm.at[pl.ds(base + g * CHUNK, CHUNK)], idx_v.at[b]
            )
            pltpu.async_copy(table_hbm.at[idx_v.at[b]], rows_v.at[b], gsem)

        def wait_gather(b):
            pltpu.make_async_copy(
                table_hbm.at[idx_v.at[b]], rows_v.at[b], gsem
            ).wait()

        def start_wb(g, b):
            pltpu.async_copy(
                rows_v.at[b], out_hbm.at[pl.ds(base + g * CHUNK, CHUNK)], wsem
            )

        def wait_wb(g, b):
            pltpu.make_async_copy(
                rows_v.at[b], out_hbm.at[pl.ds(base + g * CHUNK, CHUNK)], wsem
            ).wait()

        fetch(0, 0)
        wait_gather(0)
        start_wb(0, 0)
        fetch(1, 1)

        def pair(u, carry):
            g0 = 2 * u
            wait_gather(1)
            start_wb(g0 - 1, 1)
            wait_wb(g0 - 2, 0)
            fetch(g0, 0)
            wait_gather(0)
            start_wb(g0, 0)
            wait_wb(g0 - 1, 1)
            fetch(g0 + 1, 1)
            return carry

        lax.fori_loop(1, n_chunks // 2, pair, 0)

        wait_gather(1)
        start_wb(n_chunks - 1, 1)
        wait_wb(n_chunks - 2, 0)
        wait_wb(n_chunks - 1, 1)

    return gather_kernel


def kernel(ids, ori_weight, think_weight):
    table = jnp.concatenate([ori_weight, think_weight], axis=0)
    flat_ids = ids.reshape(-1).astype(jnp.int32)
    out = _make_gather(flat_ids.shape[0])(flat_ids, table)
    return out.reshape(ids.shape + (EMBED_DIM,))


# speedup vs baseline: 7.0089x; 1.0219x over previous
import functools

import jax
import jax.numpy as jnp
from jax import lax
from jax.experimental import pallas as pl
from jax.experimental.pallas import tpu as pltpu
from jax.experimental.pallas import tpu_sc as plsc

EMBED_DIM = 64
CHUNK = 512


def _make_gather(n_ids: int):
    info = plsc.get_sparse_core_info()
    nw = info.num_cores * info.num_subcores
    assert n_ids % (nw * 2 * CHUNK) == 0
    b_per_w = n_ids // nw
    n_chunks = b_per_w // CHUNK
    mesh = plsc.VectorSubcoreMesh(core_axis_name="c", subcore_axis_name="s")

    @functools.partial(
        pl.kernel,
        mesh=mesh,
        out_type=jax.ShapeDtypeStruct((n_ids, EMBED_DIM), jnp.float32),
        scratch_types=[
            pltpu.VMEM((n_chunks, CHUNK), jnp.int32),
            pltpu.VMEM((2, CHUNK, EMBED_DIM), jnp.float32),
            pltpu.SemaphoreType.DMA,
            pltpu.SemaphoreType.DMA,
        ],
        compiler_params=pltpu.CompilerParams(use_tc_tiling_on_sc=False),
    )
    def gather_kernel(ids_hbm, table_hbm, out_hbm, idx_v, rows_v, gsem, wsem):
        wid = lax.axis_index("s") * info.num_cores + lax.axis_index("c")
        base = wid * b_per_w

        pltpu.sync_copy(ids_hbm.at[wid], idx_v)

        def fetch(g, b):
            pltpu.async_copy(table_hbm.at[idx_v.at[g]], rows_v.at[b], gsem)

        def wait_gather(g, b):
            pltpu.make_async_copy(
                table_hbm.at[idx_v.at[g]], rows_v.at[b], gsem
            ).wait()

        def start_wb(g, b):
            pltpu.async_copy(
                rows_v.at[b], out_hbm.at[pl.ds(base + g * CHUNK, CHUNK)], wsem
            )

        def wait_wb(g, b):
            pltpu.make_async_copy(
                rows_v.at[b], out_hbm.at[pl.ds(base + g * CHUNK, CHUNK)], wsem
            ).wait()

        fetch(0, 0)
        fetch(1, 1)
        wait_gather(0, 0)
        start_wb(0, 0)

        def pair(u, carry):
            g0 = 2 * u
            wait_wb(g0 - 2, 0)
            fetch(g0, 0)
            wait_gather(g0 - 1, 1)
            start_wb(g0 - 1, 1)
            wait_wb(g0 - 1, 1)
            fetch(g0 + 1, 1)
            wait_gather(g0, 0)
            start_wb(g0, 0)
            return carry

        lax.fori_loop(1, n_chunks // 2, pair, 0)

        wait_gather(n_chunks - 1, 1)
        start_wb(n_chunks - 1, 1)
        wait_wb(n_chunks - 2, 0)
        wait_wb(n_chunks - 1, 1)

    return gather_kernel


def kernel(ids, ori_weight, think_weight):
    table = jnp.concatenate([ori_weight, think_weight], axis=0)
    n_ids = ids.shape[0] * ids.shape[1]
    info = plsc.get_sparse_core_info()
    nw = info.num_cores * info.num_subcores
    flat_ids = ids.astype(jnp.int32).reshape(nw, (n_ids // nw) // CHUNK, CHUNK)
    out = _make_gather(n_ids)(flat_ids, table)
    return out.reshape(ids.shape + (EMBED_DIM,))
